# Initial kernel scaffold; baseline (speedup 1.0000x reference)
import jax
import jax.numpy as jnp
from jax.experimental import pallas as pl


def kernel(input_mat, sparse_indices, sparse_values, dense):
    raise NotImplementedError("write your pallas kernel here")


# speedup vs baseline: 12.8422x; 12.8422x over previous
import functools

import jax
import jax.numpy as jnp
from jax import lax
from jax.experimental import pallas as pl
from jax.experimental.pallas import tpu as pltpu
from jax.experimental.pallas import tpu_sc as plsc

N = 16384
D = 64
DH = D // 2
NT = 16
CHUNK = 512
IDXW = 128
NSUB = CHUNK // IDXW


def _sc_body(nnz_pad, inp_hbm, cols_hbm, rows_hbm, vals_hbm, dflat_hbm, out_hbm,
             cols_v, rowsidx_v, vals_v, gat_v, in2_v, acc, sem):
    c = lax.axis_index("c")
    s = lax.axis_index("s")
    per_tile = nnz_pad // NT
    n_chunks = per_tile // CHUNK
    rows_per_tile = per_tile // IDXW

    def _zb(i, _):
        gat_v[i, pl.ds(0, 16)] = jnp.zeros((16,), jnp.float32)
        gat_v[i, pl.ds(16, 16)] = jnp.zeros((16,), jnp.float32)
        return _
    lax.fori_loop(0, CHUNK, _zb, None)
    arows = N // NT
    pltpu.sync_copy(gat_v, acc.at[pl.ds(s * arows, CHUNK)])
    pltpu.sync_copy(gat_v, acc.at[pl.ds(s * arows + CHUNK, CHUNK)])
    plsc.subcore_barrier()

    coff = c * N

    def _chunk(t, _):
        crow = s * rows_per_tile + t * NSUB
        base = s * per_tile + t * CHUNK
        pltpu.sync_copy(cols_hbm.at[pl.ds(crow, NSUB)], cols_v)
        pltpu.sync_copy(rows_hbm.at[pl.ds(crow, NSUB)], rowsidx_v)
        pltpu.sync_copy(vals_hbm.at[pl.ds(base, CHUNK)], vals_v)
        def _off(i, _):
            j, k = i // 8, i % 8
            cols_v[j, pl.ds(k * 16, 16)] = cols_v[j, pl.ds(k * 16, 16)] + coff
            return _
        lax.fori_loop(0, NSUB * 8, _off, None)
        cps = [pltpu.async_copy(dflat_hbm.at[cols_v.at[j]],
                                gat_v.at[pl.ds(j * IDXW, IDXW)], sem)
               for j in range(NSUB)]
        for cp in cps:
            cp.wait()
        def _mul(g, _):
            vv = vals_v[pl.ds(g * 16, 16)]
            for k in range(16):
                v = vv[k]
                i = g * 16 + k
                gat_v[i, pl.ds(0, 16)] = gat_v[i, pl.ds(0, 16)] * v
                gat_v[i, pl.ds(16, 16)] = gat_v[i, pl.ds(16, 16)] * v
            return _
        lax.fori_loop(0, CHUNK // 16, _mul, None)
        for j in range(NSUB):
            pltpu.sync_copy(gat_v.at[pl.ds(j * IDXW, IDXW)],
                            acc.at[rowsidx_v.at[j]], add=True)
        return _

    lax.fori_loop(0, n_chunks, _chunk, None)
    plsc.subcore_barrier()

    for half in range(2):
        r0 = s * arows + half * CHUNK
        pltpu.sync_copy(inp_hbm.at[c, pl.ds(r0, CHUNK)], in2_v)
        pltpu.sync_copy(acc.at[pl.ds(r0, CHUNK)], gat_v)

        def _add(i, _):
            in2_v[i, pl.ds(0, 16)] = in2_v[i, pl.ds(0, 16)] + gat_v[i, pl.ds(0, 16)]
            in2_v[i, pl.ds(16, 16)] = in2_v[i, pl.ds(16, 16)] + gat_v[i, pl.ds(16, 16)]
            return _
        lax.fori_loop(0, CHUNK, _add, None)
        pltpu.sync_copy(in2_v, out_hbm.at[c, pl.ds(r0, CHUNK)])


def kernel(input_mat, sparse_indices, sparse_values, dense):
    nnz = sparse_values.shape[0]
    tile_q = NT * CHUNK
    nnz_pad = ((nnz + tile_q - 1) // tile_q) * tile_q
    pad = nnz_pad - nnz
    rows = jnp.pad(sparse_indices[0], (0, pad)).reshape(nnz_pad // IDXW, IDXW)
    cols = jnp.pad(sparse_indices[1], (0, pad)).reshape(nnz_pad // IDXW, IDXW)
    vals = jnp.pad(sparse_values, (0, pad))
    dflat = jnp.concatenate([dense[:, :DH], dense[:, DH:]], axis=0)
    inp2 = jnp.stack([input_mat[:, :DH], input_mat[:, DH:]])

    mesh = plsc.VectorSubcoreMesh(core_axis_name="c", subcore_axis_name="s")
    body = functools.partial(_sc_body, nnz_pad)
    out2 = pl.kernel(
        body,
        out_type=jax.ShapeDtypeStruct((2, N, DH), jnp.float32),
        mesh=mesh,
        compiler_params=pltpu.CompilerParams(use_tc_tiling_on_sc=False),
        scratch_types=[
            pltpu.VMEM((NSUB, IDXW), jnp.int32),
            pltpu.VMEM((NSUB, IDXW), jnp.int32),
            pltpu.VMEM((CHUNK,), jnp.float32),
            pltpu.VMEM((CHUNK, DH), jnp.float32),
            pltpu.VMEM((CHUNK, DH), jnp.float32),
            pltpu.VMEM_SHARED((N, DH), jnp.float32),
            pltpu.SemaphoreType.DMA,
        ],
    )(inp2, cols, rows, vals, dflat)
    return jnp.concatenate([out2[0], out2[1]], axis=1)


# speedup vs baseline: 24.1450x; 1.8801x over previous
import functools

import jax
import jax.numpy as jnp
from jax import lax
from jax.experimental import pallas as pl
from jax.experimental.pallas import tpu as pltpu
from jax.experimental.pallas import tpu_sc as plsc

N = 16384
D = 64
DH = D // 2
NT = 16
CHUNK = 512
IDXW = 128
NSUB = CHUNK // IDXW
SUP = 8
SROWS = SUP * CHUNK // IDXW


def _sc_body(nsup, inp_hbm, p_hbm, dflat_hbm, out_hbm,
             pbuf, gat, acc, sem_p, sem_g, sem_s):
    c = lax.axis_index("c")
    s = lax.axis_index("s")
    coff = c * N
    tile_rows = nsup * SROWS

    def _zb(i, _):
        gat[0, i, pl.ds(0, 16)] = jnp.zeros((16,), jnp.float32)
        gat[0, i, pl.ds(16, 16)] = jnp.zeros((16,), jnp.float32)
        return _
    lax.fori_loop(0, CHUNK, _zb, None)
    arows = N // NT
    pltpu.sync_copy(gat.at[0], acc.at[pl.ds(s * arows, CHUNK)])
    pltpu.sync_copy(gat.at[0], acc.at[pl.ds(s * arows + CHUNK, CHUNK)])
    plsc.subcore_barrier()

    def _p_slice(u):
        return p_hbm.at[pl.ds(s * tile_rows + u * SROWS, SROWS)]

    def _sup(u, b):
        pltpu.make_async_copy(_p_slice(u), pbuf.at[b], sem_p.at[b]).wait()
        un = jnp.minimum(u + 1, nsup - 1)
        pltpu.async_copy(_p_slice(un), pbuf.at[1 - b], sem_p.at[1 - b])

        pend_g, pend_s = {}, {}

        def fire_gather(k):
            g = k % 2
            def _off(i, _):
                r = k * NSUB + i // 8
                l = (i % 8) * 16
                pbuf[b, r, 0, pl.ds(l, 16)] = pbuf[b, r, 0, pl.ds(l, 16)] + coff
                return _
            lax.fori_loop(0, CHUNK // 16, _off, None)
            pend_g[k] = [
                pltpu.async_copy(dflat_hbm.at[pbuf.at[b, k * NSUB + j, 0]],
                                 gat.at[g, pl.ds(j * IDXW, IDXW)], sem_g.at[g])
                for j in range(NSUB)]

        def scale_scatter(k):
            g = k % 2
            for cp in pend_g.pop(k):
                cp.wait()
            def _mul(i, _):
                r = k * NSUB + i // 8
                l = (i % 8) * 16
                vv = plsc.bitcast(pbuf[b, r, 2, pl.ds(l, 16)], jnp.float32)
                for t in range(16):
                    q = i * 16 + t
                    gat[g, q, pl.ds(0, 16)] = gat[g, q, pl.ds(0, 16)] * vv[t]
                    gat[g, q, pl.ds(16, 16)] = gat[g, q, pl.ds(16, 16)] * vv[t]
                return _
            lax.fori_loop(0, CHUNK // 16, _mul, None)
            pend_s[k] = [
                pltpu.async_copy(gat.at[g, pl.ds(j * IDXW, IDXW)],
                                 acc.at[pbuf.at[b, k * NSUB + j, 1]],
                                 sem_s.at[g], add=True)
                for j in range(NSUB)]

        for k in range(SUP):
            if k >= 2:
                for cp in pend_s.pop(k - 2):
                    cp.wait()
            fire_gather(k)
            if k >= 1:
                scale_scatter(k - 1)
        scale_scatter(SUP - 1)
        for kk in (SUP - 2, SUP - 1):
            for cp in pend_s.pop(kk):
                cp.wait()

    pltpu.async_copy(_p_slice(0), pbuf.at[0], sem_p.at[0])

    def _pair(u2, _):
        _sup(2 * u2, 0)
        _sup(2 * u2 + 1, 1)
        return _
    lax.fori_loop(0, nsup // 2, _pair, None)
    if nsup % 2:
        _sup(nsup - 1, 0)
    last_pend = 1 - ((nsup - 1) % 2)
    pltpu.make_async_copy(_p_slice(nsup - 1), pbuf.at[last_pend],
                          sem_p.at[last_pend]).wait()

    plsc.subcore_barrier()

    for half in range(2):
        r0 = s * arows + half * CHUNK
        pltpu.sync_copy(inp_hbm.at[c, pl.ds(r0, CHUNK)], gat.at[0])
        pltpu.sync_copy(acc.at[pl.ds(r0, CHUNK)], gat.at[1])

        def _add(i, _):
            gat[0, i, pl.ds(0, 16)] = gat[0, i, pl.ds(0, 16)] + gat[1, i, pl.ds(0, 16)]
            gat[0, i, pl.ds(16, 16)] = gat[0, i, pl.ds(16, 16)] + gat[1, i, pl.ds(16, 16)]
            return _
        lax.fori_loop(0, CHUNK, _add, None)
        pltpu.sync_copy(gat.at[0], out_hbm.at[c, pl.ds(r0, CHUNK)])


def kernel(input_mat, sparse_indices, sparse_values, dense):
    nnz = sparse_values.shape[0]
    quantum = NT * SUP * CHUNK
    nnz_pad = ((nnz + quantum - 1) // quantum) * quantum
    nsup = nnz_pad // quantum
    pad = nnz_pad - nnz
    ar = jnp.arange(pad, dtype=jnp.int32)
    rows_p = jnp.concatenate([sparse_indices[0], (ar * 97) % N])
    cols_p = jnp.concatenate([sparse_indices[1], (ar * 89) % N])
    vals_p = jnp.pad(sparse_values, (0, pad))
    pmeta = jnp.stack([
        cols_p.reshape(-1, IDXW),
        rows_p.reshape(-1, IDXW),
        lax.bitcast_convert_type(vals_p, jnp.int32).reshape(-1, IDXW),
    ], axis=1)
    dflat = jnp.concatenate([dense[:, :DH], dense[:, DH:]], axis=0)
    inp2 = jnp.stack([input_mat[:, :DH], input_mat[:, DH:]])

    mesh = plsc.VectorSubcoreMesh(core_axis_name="c", subcore_axis_name="s")
    body = functools.partial(_sc_body, nsup)
    out2 = pl.kernel(
        body,
        out_type=jax.ShapeDtypeStruct((2, N, DH), jnp.float32),
        mesh=mesh,
        compiler_params=pltpu.CompilerParams(use_tc_tiling_on_sc=False,
                                             needs_layout_passes=False),
        scratch_types=[
            pltpu.VMEM((2, SROWS, 3, IDXW), jnp.int32),
            pltpu.VMEM((2, CHUNK, DH), jnp.float32),
            pltpu.VMEM_SHARED((N, DH), jnp.float32),
            pltpu.SemaphoreType.DMA((2,)),
            pltpu.SemaphoreType.DMA((2,)),
            pltpu.SemaphoreType.DMA((2,)),
        ],
    )(inp2, pmeta, dflat)
    return jnp.concatenate([out2[0], out2[1]], axis=1)
